# Initial kernel scaffold; baseline (speedup 1.0000x reference)
import jax
import jax.numpy as jnp
from jax.experimental import pallas as pl


def kernel(sequence, token_table, genre_table, token_to_genres):
    raise NotImplementedError("write your pallas kernel here")


# speedup vs baseline: 17.3944x; 17.3944x over previous
import functools

import jax
import jax.numpy as jnp
import numpy as np
from jax import lax
from jax.experimental import pallas as pl
from jax.experimental.pallas import tpu as pltpu
from jax.experimental.pallas import tpu_sc as plsc

VOCAB = 100000
D = 64
MAXLEN = 200
NG1 = 21
MAX_G = 3
BATCH = 4096
SEQLEN = 200
N = BATCH * SEQLEN


def _pe_table():
    pe = np.zeros((MAXLEN, D), dtype=np.float32)
    position = np.arange(MAXLEN, dtype=np.float32)[:, None]
    div_term = np.exp(np.arange(0, D, 2, dtype=np.float32) * (-np.log(10000.0) / D))
    pe[:, 0::2] = np.sin(position * div_term)
    pe[:, 1::2] = np.cos(position * div_term)
    return pe


_PE = _pe_table()


_R = 2000


def _fuse_body(tok_ref, gid_ref, gtab_ref, out_ref):
    gids = gid_ref[...]
    iota = lax.broadcasted_iota(jnp.int32, (_R, NG1), 1)
    counts = jnp.zeros((_R, NG1), jnp.float32)
    for g in range(MAX_G):
        gid_g = lax.slice(gids, (0, g), (_R, g + 1))
        counts = counts + (gid_g == iota).astype(jnp.float32)
    gavg = lax.dot_general(
        counts, gtab_ref[...], (((1,), (0,)), ((), ())),
        preferred_element_type=jnp.float32,
    )
    out_ref[...] = tok_ref[...] + gavg * (1.0 / MAX_G)


def _build_fused(token_table, genre_table, token_to_genres):
    return pl.pallas_call(
        _fuse_body,
        grid=(VOCAB // _R,),
        in_specs=[
            pl.BlockSpec((_R, D), lambda i: (i, 0)),
            pl.BlockSpec((_R, MAX_G), lambda i: (i, 0)),
            pl.BlockSpec((NG1, D), lambda i: (0, 0)),
        ],
        out_specs=pl.BlockSpec((_R, D), lambda i: (i, 0)),
        out_shape=jax.ShapeDtypeStruct((VOCAB, D), jnp.float32),
    )(token_table, token_to_genres, genre_table)


_NW = 32
_PER_W = N // _NW
_CH = 200
_NCH = _PER_W // _CH
_IW = 100

def _gather_pe_body(fused_hbm, seqr_hbm, pe_hbm, out_hbm, idx_v, rows_v, pe_v, sem):
    wid = lax.axis_index("s") * 2 + lax.axis_index("c")
    base = wid * _PER_W
    irow0 = wid * (_PER_W // _IW)
    pltpu.sync_copy(pe_hbm, pe_v)

    @pl.loop(0, _NCH)
    def _(c):
        pltpu.sync_copy(seqr_hbm.at[pl.ds(irow0 + c * 2, 2)], idx_v)
        cp0 = pltpu.async_copy(
            fused_hbm.at[idx_v.at[0]], rows_v.at[pl.ds(0, _IW)], sem)
        cp1 = pltpu.async_copy(
            fused_hbm.at[idx_v.at[1]], rows_v.at[pl.ds(_IW, _IW)], sem)
        cp0.wait()
        cp1.wait()

        @pl.loop(0, _CH)
        def _(j):
            for s in range(D // 16):
                sl = pl.ds(s * 16, 16)
                rows_v[j, sl] = rows_v[j, sl] + pe_v[j, sl]

        pltpu.sync_copy(rows_v, out_hbm.at[pl.ds(base + c * _CH, _CH)])


@functools.cache
def _gather_pe():
    mesh = plsc.VectorSubcoreMesh(core_axis_name="c", subcore_axis_name="s")
    return pl.kernel(
        _gather_pe_body,
        out_type=jax.ShapeDtypeStruct((N, D), jnp.float32),
        mesh=mesh,
        scratch_types=[
            pltpu.VMEM((2, _IW), jnp.int32),
            pltpu.VMEM((_CH, D), jnp.float32),
            pltpu.VMEM((MAXLEN, D), jnp.float32),
            pltpu.SemaphoreType.DMA,
        ],
        compiler_params=pltpu.CompilerParams(use_tc_tiling_on_sc=False),
    )


def kernel(sequence, token_table, genre_table, token_to_genres):
    fused = _build_fused(token_table, genre_table, token_to_genres)
    seq_r = sequence.reshape(N // _IW, _IW)
    pe = jnp.asarray(_PE)
    out = _gather_pe()(fused, seq_r, pe)
    return out.reshape(BATCH, SEQLEN, D)
